# baseline (speedup 1.0000x reference)
import functools

import jax
import jax.numpy as jnp
from jax import lax
from jax.experimental import pallas as pl
from jax.experimental.pallas import tpu as pltpu
from jax.experimental.pallas import tpu_sc as plsc


def _make_sc_gather(B, T, I, E, K):
    N = B * E * K
    NW = 32
    CH = N // NW
    EK = E * K
    ROWS_PER_B = T * I // 128
    mesh = plsc.VectorSubcoreMesh(core_axis_name="c", subcore_axis_name="s")

    @functools.partial(
        pl.kernel,
        out_type=jax.ShapeDtypeStruct((N,), jnp.float32),
        mesh=mesh,
        scratch_types=[
            pltpu.VMEM((CH,), jnp.int32),
            pltpu.VMEM((CH, 128), jnp.float32),
            pltpu.VMEM((CH,), jnp.float32),
            pltpu.SemaphoreType.DMA,
        ],
    )
    def sc_gather(x_hbm, ind_hbm, out_hbm, idx_v, rows_v, val_v, sem):
        wid = lax.axis_index("s") * 2 + lax.axis_index("c")
        base = wid * CH
        b = base // EK
        pltpu.sync_copy(ind_hbm.at[pl.ds(base, CH)], idx_v)
        for i in range(CH // 16):
            t = idx_v[pl.ds(i * 16, 16)]
            idx_v[pl.ds(i * 16, 16)] = b * ROWS_PER_B + t * (I // 128)
        copies = []
        for c in range(CH // 128):
            copies.append(
                pltpu.async_copy(
                    x_hbm.at[idx_v.at[pl.ds(c * 128, 128)]],
                    rows_v.at[pl.ds(c * 128, 128)],
                    sem,
                )
            )
        for cp in copies:
            cp.wait()
        lane = lax.iota(jnp.int32, 16)
        zero16 = jnp.zeros((16, 1), jnp.int32)
        dnums = lax.GatherDimensionNumbers(
            offset_dims=(), collapsed_slice_dims=(0,), start_index_map=(0,))
        for i in range(CH // 16):
            acc = jnp.zeros((16,), jnp.float32)
            for r in range(16):
                v = rows_v[i * 16 + r, pl.ds(0, 16)]
                bcast = lax.gather(
                    v, zero16, dnums, slice_sizes=(1,),
                    mode=lax.GatherScatterMode.PROMISE_IN_BOUNDS)
                acc = jnp.where(lane == r, bcast, acc)
            val_v[pl.ds(i * 16, 16)] = acc
        pltpu.sync_copy(val_v, out_hbm.at[pl.ds(base, CH)])

    return sc_gather


def _make_tc_reduce(E, I, J, RB):
    EPB = RB // I

    def body(w_ref, ws_ref):
        w = w_ref[...]
        parts = [
            jnp.sum(w[n * I:(n + 1) * I], axis=0, keepdims=True)
            for n in range(EPB)
        ]
        ws_ref[...] = jnp.concatenate(parts, axis=0)[:, None, :]

    return pl.pallas_call(
        body,
        grid=(E // EPB,),
        in_specs=[pl.BlockSpec((RB, J), lambda r: (r, 0))],
        out_specs=pl.BlockSpec((EPB, 1, J), lambda r: (r, 0, 0)),
        out_shape=jax.ShapeDtypeStruct((E, 1, J), jnp.float32),
        compiler_params=pltpu.CompilerParams(
            dimension_semantics=("arbitrary",),
        ),
    )


def _make_tc_broadcast(B, E, K, J, EB):
    def body(xg_ref, ws_ref, y_ref):
        xg = xg_ref[0]
        ws = ws_ref[:, 0, :]
        y_ref[0] = xg[:, :, None] * ws[:, None, :]

    return pl.pallas_call(
        body,
        grid=(B, E // EB),
        in_specs=[
            pl.BlockSpec((1, EB, K), lambda b, e: (b, e, 0)),
            pl.BlockSpec((EB, 1, J), lambda b, e: (e, 0, 0)),
        ],
        out_specs=pl.BlockSpec((1, EB, K, J), lambda b, e: (b, e, 0, 0)),
        out_shape=jax.ShapeDtypeStruct((B, E, K, J), jnp.float32),
        compiler_params=pltpu.CompilerParams(
            dimension_semantics=("arbitrary", "arbitrary"),
        ),
    )


def kernel(X, ind, W):
    B, T, I = X.shape
    E, K = ind.shape[1], ind.shape[2]
    J = W.shape[2]

    x_rows = X.reshape(-1, 128)
    xg_flat = _make_sc_gather(B, T, I, E, K)(x_rows, ind.reshape(-1))
    xg = xg_flat.reshape(B, E, K)

    wsum = _make_tc_reduce(E, I, J, RB=2 * I)(W.reshape(E * I, J))
    return _make_tc_broadcast(B, E, K, J, EB=8)(xg, wsum)


# speedup vs baseline: 1.5911x; 1.5911x over previous
import functools

import jax
import jax.numpy as jnp
from jax import lax
from jax.experimental import pallas as pl
from jax.experimental.pallas import tpu as pltpu
from jax.experimental.pallas import tpu_sc as plsc


def _make_sc_gather(B, T, E, K):
    N = B * E * K
    NW = 32
    CH = N // NW
    RPW = CH // K
    EK = E * K
    mesh = plsc.VectorSubcoreMesh(core_axis_name="c", subcore_axis_name="s")

    @functools.partial(
        pl.kernel,
        out_type=jax.ShapeDtypeStruct((N,), jnp.float32),
        mesh=mesh,
        scratch_types=[
            pltpu.VMEM((RPW, K), jnp.int32),
            pltpu.VMEM((CH,), jnp.float32),
            pltpu.SemaphoreType.DMA,
        ],
    )
    def sc_gather(x0_hbm, ind_hbm, out_hbm, idx_v, val_v, sem):
        wid = lax.axis_index("s") * 2 + lax.axis_index("c")
        base = wid * CH
        b = base // EK
        pltpu.sync_copy(ind_hbm.at[pl.ds(wid * RPW, RPW)], idx_v)
        for r in range(RPW):
            for i in range(K // 16):
                t = idx_v[r, pl.ds(i * 16, 16)]
                idx_v[r, pl.ds(i * 16, 16)] = t + b * T
        copies = []
        for r in range(RPW):
            copies.append(
                pltpu.async_copy(
                    x0_hbm.at[idx_v.at[r]],
                    val_v.at[pl.ds(r * K, K)],
                    sem,
                )
            )
        for cp in copies:
            cp.wait()
        pltpu.sync_copy(val_v, out_hbm.at[pl.ds(base, CH)])

    return sc_gather


def _make_tc_reduce(E, I, J, RB):
    EPB = RB // I

    def body(w_ref, ws_ref):
        w = w_ref[...]
        parts = [
            jnp.sum(w[n * I:(n + 1) * I], axis=0, keepdims=True)
            for n in range(EPB)
        ]
        ws_ref[...] = jnp.concatenate(parts, axis=0)[:, None, :]

    return pl.pallas_call(
        body,
        grid=(E * I // RB,),
        in_specs=[pl.BlockSpec((RB, J), lambda r: (r, 0))],
        out_specs=pl.BlockSpec((EPB, 1, J), lambda r: (r, 0, 0)),
        out_shape=jax.ShapeDtypeStruct((E, 1, J), jnp.float32),
        compiler_params=pltpu.CompilerParams(
            dimension_semantics=("arbitrary",),
        ),
    )


def _make_tc_broadcast(B, E, K, J, EB):
    def body(xg_ref, ws_ref, y_ref):
        xg = xg_ref[0]
        ws = ws_ref[:, 0, :]
        y_ref[0] = xg[:, :, None] * ws[:, None, :]

    return pl.pallas_call(
        body,
        grid=(B, E // EB),
        in_specs=[
            pl.BlockSpec((1, EB, K), lambda b, e: (b, e, 0)),
            pl.BlockSpec((EB, 1, J), lambda b, e: (e, 0, 0)),
        ],
        out_specs=pl.BlockSpec((1, EB, K, J), lambda b, e: (b, e, 0, 0)),
        out_shape=jax.ShapeDtypeStruct((B, E, K, J), jnp.float32),
        compiler_params=pltpu.CompilerParams(
            dimension_semantics=("arbitrary", "arbitrary"),
        ),
    )


def kernel(X, ind, W):
    B, T, I = X.shape
    E, K = ind.shape[1], ind.shape[2]
    J = W.shape[2]

    x0 = X[:, :, 0].reshape(-1)
    xg_flat = _make_sc_gather(B, T, E, K)(x0, ind.reshape(B * E, K))
    xg = xg_flat.reshape(B, E, K)

    wsum = _make_tc_reduce(E, I, J, RB=4 * I)(W.reshape(E * I, J))
    return _make_tc_broadcast(B, E, K, J, EB=8)(xg, wsum)
